# baseline (speedup 1.0000x reference)
import functools

import jax
import jax.numpy as jnp
from jax import lax
from jax.experimental import pallas as pl
from jax.experimental.pallas import tpu as pltpu
from jax.experimental.pallas import tpu_sc as plsc

N = 10000
D = 128
E = 320000
NC, NS = 2, 16
NW = NC * NS
N_PAD = 10240
RPT = N_PAD // NS
CHUNK = 320
FAST_CID = 1
ECF = 32
ECS = 32
ECP = ECF + 4
E_PAD = NS * (ECF + ECS) * CHUNK

_mesh = plsc.VectorSubcoreMesh(core_axis_name="c", subcore_axis_name="s")


@functools.partial(
    pl.kernel,
    mesh=_mesh,
    out_type=jax.ShapeDtypeStruct((NC, N_PAD), jnp.float32),
    scratch_types=[
        pltpu.VMEM((ECP, 2, 1, CHUNK), jnp.int32),
        pltpu.VMEM((1, CHUNK), jnp.float32),
        pltpu.VMEM_SHARED((N_PAD,), jnp.float32),
    ],
)
def _deg_kernel(ei_hbm, zeros_hbm, out_hbm, idx_v, ones_v, acc):
    cid = lax.axis_index("c")
    sid = lax.axis_index("s")
    wid = cid * NS + sid
    for i in range(CHUNK // 16):
        ones_v[0, pl.ds(i * 16, 16)] = jnp.ones((16,), jnp.float32)
    pltpu.sync_copy(zeros_hbm.at[pl.ds(sid * RPT, RPT)],
                    acc.at[pl.ds(sid * RPT, RPT)])
    pltpu.sync_copy(ei_hbm.at[wid], idx_v)
    plsc.subcore_barrier()

    def body(j, c):
        pltpu.sync_copy(ones_v.at[0], acc.at[idx_v.at[j, 1, 0]], add=True)
        return c

    lax.fori_loop(0, jnp.where(cid == FAST_CID, ECF, ECS), body, 0)
    plsc.subcore_barrier()
    pltpu.sync_copy(acc.at[pl.ds(sid * RPT, RPT)],
                    out_hbm.at[cid, pl.ds(sid * RPT, RPT)])


@functools.partial(
    pl.kernel,
    mesh=_mesh,
    out_type=jax.ShapeDtypeStruct((NC, N_PAD, D), jnp.float32),
    scratch_types=[
        pltpu.VMEM((4, 2, 1, CHUNK), jnp.int32),
        pltpu.VMEM((CHUNK, D), jnp.float32),
        pltpu.VMEM_SHARED((N_PAD, D), jnp.float32),
        pltpu.SemaphoreType.DMA,
        pltpu.SemaphoreType.DMA,
        pltpu.SemaphoreType.DMA,
        pltpu.SemaphoreType.DMA,
        pltpu.SemaphoreType.DMA,
    ],
)
def _agg_kernel(y_hbm, ei_hbm, out_hbm, ring, rows_v, acc,
                isem0, isem1, isem2, isem3, gsem):
    cid = lax.axis_index("c")
    sid = lax.axis_index("s")
    wid = cid * NS + sid
    isems = (isem0, isem1, isem2, isem3)
    pltpu.sync_copy(y_hbm.at[pl.ds(sid * RPT, RPT)],
                    acc.at[pl.ds(sid * RPT, RPT)])
    plsc.subcore_barrier()

    for s in range(4):
        pltpu.async_copy(ei_hbm.at[wid, s], ring.at[s], isems[s])

    def body(i, c):
        for b in range(4):
            j = 4 * i + b
            pltpu.make_async_copy(ei_hbm.at[wid, j], ring.at[b],
                                  isems[b]).wait()
            pltpu.async_copy(y_hbm.at[ring.at[b, 0, 0]], rows_v, gsem).wait()
            pltpu.sync_copy(rows_v, acc.at[ring.at[b, 1, 0]], add=True)
            pltpu.async_copy(ei_hbm.at[wid, j + 4], ring.at[b], isems[b])
        return c

    lax.fori_loop(0, jnp.where(cid == FAST_CID, ECF // 4, ECS // 4), body, 0)
    for s in range(4):
        pltpu.make_async_copy(ei_hbm.at[wid, 0], ring.at[s], isems[s]).wait()
    plsc.subcore_barrier()
    pltpu.sync_copy(acc.at[pl.ds(sid * RPT, RPT)],
                    out_hbm.at[cid, pl.ds(sid * RPT, RPT)])


def _prep_body(degp_ref, o_ref):
    deg = degp_ref[0] + degp_ref[1] + 1.0
    dv = lax.rsqrt(deg)
    row = lax.broadcasted_iota(jnp.int32, (N_PAD // 128, 128), 0)
    col = lax.broadcasted_iota(jnp.int32, (N_PAD // 128, 128), 1)
    o_ref[...] = jnp.where(row * 128 + col < N, dv, 0.0)


_prep_call = pl.pallas_call(
    _prep_body,
    out_shape=jax.ShapeDtypeStruct((N_PAD // 128, 128), jnp.float32),
)

BR = 256
GRID = N_PAD // BR


def _y1_body(x_ref, dinv_ref, w_ref, o_ref):
    o_ref[...] = jnp.dot(x_ref[...] * dinv_ref[...], w_ref[...],
                         preferred_element_type=jnp.float32)


_y1_call = pl.pallas_call(
    _y1_body,
    grid=(GRID,),
    in_specs=[
        pl.BlockSpec((BR, D), lambda i: (i, 0)),
        pl.BlockSpec((BR, 1), lambda i: (i, 0)),
        pl.BlockSpec((D, D), lambda i: (0, 0)),
    ],
    out_specs=pl.BlockSpec((BR, D), lambda i: (i, 0)),
    out_shape=jax.ShapeDtypeStruct((N_PAD, D), jnp.float32),
)


def _mid_body(a_ref, yp_ref, dinv_ref, b_ref, w_ref, o_ref):
    t = a_ref[0] + a_ref[1] - yp_ref[...]
    t = t * dinv_ref[...] + b_ref[...]
    t = jnp.maximum(t, 0.0)
    o_ref[...] = jnp.dot(t * dinv_ref[...], w_ref[...],
                         preferred_element_type=jnp.float32)


_mid_call = pl.pallas_call(
    _mid_body,
    grid=(GRID,),
    in_specs=[
        pl.BlockSpec((NC, BR, D), lambda i: (0, i, 0)),
        pl.BlockSpec((BR, D), lambda i: (i, 0)),
        pl.BlockSpec((BR, 1), lambda i: (i, 0)),
        pl.BlockSpec((1, D), lambda i: (0, 0)),
        pl.BlockSpec((D, D), lambda i: (0, 0)),
    ],
    out_specs=pl.BlockSpec((BR, D), lambda i: (i, 0)),
    out_shape=jax.ShapeDtypeStruct((N_PAD, D), jnp.float32),
)


def _fin_body(a_ref, yp_ref, dinv_ref, b_ref, o_ref):
    t = a_ref[0] + a_ref[1] - yp_ref[...]
    o_ref[...] = t * dinv_ref[...] + b_ref[...]


_fin_call = pl.pallas_call(
    _fin_body,
    grid=(GRID,),
    in_specs=[
        pl.BlockSpec((NC, BR, D), lambda i: (0, i, 0)),
        pl.BlockSpec((BR, D), lambda i: (i, 0)),
        pl.BlockSpec((BR, 1), lambda i: (i, 0)),
        pl.BlockSpec((1, D), lambda i: (0, 0)),
    ],
    out_specs=pl.BlockSpec((BR, D), lambda i: (i, 0)),
    out_shape=jax.ShapeDtypeStruct((N_PAD, D), jnp.float32),
)


def kernel(x, edge_index, W1, b1, W2, b2, W3, b3):
    src = edge_index[0].astype(jnp.int32)
    dst = edge_index[1].astype(jnp.int32)
    pad = jnp.full((E_PAD - E,), N, jnp.int32)
    srcf = jnp.concatenate([src, pad])
    dstf = jnp.concatenate([dst, pad])

    def _tile_block(nchunks, off):
        n = NS * nchunks * CHUNK
        blk = jnp.stack(
            [srcf[off:off + n].reshape(NS, nchunks, 1, CHUNK),
             dstf[off:off + n].reshape(NS, nchunks, 1, CHUNK)], axis=2)
        fill = jnp.full((NS, ECP - nchunks, 2, 1, CHUNK), N, jnp.int32)
        return jnp.concatenate([blk, fill], axis=1)

    fast_blk = _tile_block(ECF, 0)
    slow_blk = _tile_block(ECS, NS * ECF * CHUNK)
    eip = jnp.concatenate(
        [fast_blk, slow_blk] if FAST_CID == 0 else [slow_blk, fast_blk],
        axis=0)
    xp = jnp.pad(x, ((0, N_PAD - N), (0, 0)))
    zeros = jnp.zeros((N_PAD,), jnp.float32)

    degp = _deg_kernel(eip, zeros)
    dinv = _prep_call(degp.reshape(NC, N_PAD // 128, 128)).reshape(N_PAD, 1)

    b1r, b2r, b3r = (b.reshape(1, D) for b in (b1, b2, b3))
    y1 = _y1_call(xp, dinv, W1)
    a1 = _agg_kernel(y1, eip)
    y2 = _mid_call(a1, y1, dinv, b1r, W2)
    a2 = _agg_kernel(y2, eip)
    y3 = _mid_call(a2, y2, dinv, b2r, W3)
    a3 = _agg_kernel(y3, eip)
    out = _fin_call(a3, y3, dinv, b3r)
    return out[:N]


# speedup vs baseline: 1.0381x; 1.0381x over previous
import functools

import jax
import jax.numpy as jnp
from jax import lax
from jax.experimental import pallas as pl
from jax.experimental.pallas import tpu as pltpu
from jax.experimental.pallas import tpu_sc as plsc

N = 10000
D = 128
E = 320000
NC, NS = 2, 16
NW = NC * NS
N_PAD = 10240
RPT = N_PAD // NS
CHUNK = 256
FAST_CID = 1
ECF = 40
ECS = 40
ECP = ECF + 4
E_PAD = NS * (ECF + ECS) * CHUNK

_mesh = plsc.VectorSubcoreMesh(core_axis_name="c", subcore_axis_name="s")


@functools.partial(
    pl.kernel,
    mesh=_mesh,
    out_type=jax.ShapeDtypeStruct((NC, N_PAD), jnp.float32),
    scratch_types=[
        pltpu.VMEM((ECP, 2, 1, CHUNK), jnp.int32),
        pltpu.VMEM((1, CHUNK), jnp.float32),
        pltpu.VMEM_SHARED((N_PAD,), jnp.float32),
    ],
)
def _deg_kernel(ei_hbm, zeros_hbm, out_hbm, idx_v, ones_v, acc):
    cid = lax.axis_index("c")
    sid = lax.axis_index("s")
    wid = cid * NS + sid
    for i in range(CHUNK // 16):
        ones_v[0, pl.ds(i * 16, 16)] = jnp.ones((16,), jnp.float32)
    pltpu.sync_copy(zeros_hbm.at[pl.ds(sid * RPT, RPT)],
                    acc.at[pl.ds(sid * RPT, RPT)])
    pltpu.sync_copy(ei_hbm.at[wid], idx_v)
    plsc.subcore_barrier()

    def body(j, c):
        pltpu.sync_copy(ones_v.at[0], acc.at[idx_v.at[j, 1, 0]], add=True)
        return c

    lax.fori_loop(0, jnp.where(cid == FAST_CID, ECF, ECS), body, 0)
    plsc.subcore_barrier()
    pltpu.sync_copy(acc.at[pl.ds(sid * RPT, RPT)],
                    out_hbm.at[cid, pl.ds(sid * RPT, RPT)])


@functools.partial(
    pl.kernel,
    mesh=_mesh,
    out_type=jax.ShapeDtypeStruct((NC, N_PAD, D), jnp.float32),
    scratch_types=[
        pltpu.VMEM((4, 2, 1, CHUNK), jnp.int32),
        pltpu.VMEM((CHUNK, D), jnp.float32),
        pltpu.VMEM_SHARED((N_PAD, D), jnp.float32),
        pltpu.SemaphoreType.DMA,
        pltpu.SemaphoreType.DMA,
        pltpu.SemaphoreType.DMA,
        pltpu.SemaphoreType.DMA,
        pltpu.SemaphoreType.DMA,
        pltpu.SemaphoreType.DMA,
    ],
)
def _agg_kernel(y_hbm, ei_hbm, out_hbm, ring, rows_v, acc,
                isem0, isem1, isem2, isem3, gsem, gsem2):
    cid = lax.axis_index("c")
    sid = lax.axis_index("s")
    wid = cid * NS + sid
    isems = (isem0, isem1, isem2, isem3)
    pltpu.sync_copy(y_hbm.at[pl.ds(sid * RPT, RPT)],
                    acc.at[pl.ds(sid * RPT, RPT)])
    plsc.subcore_barrier()

    for s in range(4):
        pltpu.async_copy(ei_hbm.at[wid, s], ring.at[s], isems[s])

    def body(i, c):
        for b in range(4):
            j = 4 * i + b
            pltpu.make_async_copy(ei_hbm.at[wid, j], ring.at[b],
                                  isems[b]).wait()
            h1 = pltpu.async_copy(y_hbm.at[ring.at[b, 0, 0, pl.ds(0, 128)]],
                                  rows_v.at[pl.ds(0, 128)], gsem)
            h2 = pltpu.async_copy(y_hbm.at[ring.at[b, 0, 0, pl.ds(128, 128)]],
                                  rows_v.at[pl.ds(128, 128)], gsem2)
            h1.wait()
            h2.wait()
            pltpu.sync_copy(rows_v, acc.at[ring.at[b, 1, 0]], add=True)
            pltpu.async_copy(ei_hbm.at[wid, j + 4], ring.at[b], isems[b])
        return c

    lax.fori_loop(0, jnp.where(cid == FAST_CID, ECF // 4, ECS // 4), body, 0)
    for s in range(4):
        pltpu.make_async_copy(ei_hbm.at[wid, 0], ring.at[s], isems[s]).wait()
    plsc.subcore_barrier()
    pltpu.sync_copy(acc.at[pl.ds(sid * RPT, RPT)],
                    out_hbm.at[cid, pl.ds(sid * RPT, RPT)])


def _prep_body(degp_ref, o_ref):
    deg = degp_ref[0] + degp_ref[1] + 1.0
    dv = lax.rsqrt(deg)
    row = lax.broadcasted_iota(jnp.int32, (N_PAD // 128, 128), 0)
    col = lax.broadcasted_iota(jnp.int32, (N_PAD // 128, 128), 1)
    o_ref[...] = jnp.where(row * 128 + col < N, dv, 0.0)


_prep_call = pl.pallas_call(
    _prep_body,
    out_shape=jax.ShapeDtypeStruct((N_PAD // 128, 128), jnp.float32),
)

BR = 256
GRID = N_PAD // BR


def _y1_body(x_ref, dinv_ref, w_ref, o_ref):
    o_ref[...] = jnp.dot(x_ref[...] * dinv_ref[...], w_ref[...],
                         preferred_element_type=jnp.float32)


_y1_call = pl.pallas_call(
    _y1_body,
    grid=(GRID,),
    in_specs=[
        pl.BlockSpec((BR, D), lambda i: (i, 0)),
        pl.BlockSpec((BR, 1), lambda i: (i, 0)),
        pl.BlockSpec((D, D), lambda i: (0, 0)),
    ],
    out_specs=pl.BlockSpec((BR, D), lambda i: (i, 0)),
    out_shape=jax.ShapeDtypeStruct((N_PAD, D), jnp.float32),
)


def _mid_body(a_ref, yp_ref, dinv_ref, b_ref, w_ref, o_ref):
    t = a_ref[0] + a_ref[1] - yp_ref[...]
    t = t * dinv_ref[...] + b_ref[...]
    t = jnp.maximum(t, 0.0)
    o_ref[...] = jnp.dot(t * dinv_ref[...], w_ref[...],
                         preferred_element_type=jnp.float32)


_mid_call = pl.pallas_call(
    _mid_body,
    grid=(GRID,),
    in_specs=[
        pl.BlockSpec((NC, BR, D), lambda i: (0, i, 0)),
        pl.BlockSpec((BR, D), lambda i: (i, 0)),
        pl.BlockSpec((BR, 1), lambda i: (i, 0)),
        pl.BlockSpec((1, D), lambda i: (0, 0)),
        pl.BlockSpec((D, D), lambda i: (0, 0)),
    ],
    out_specs=pl.BlockSpec((BR, D), lambda i: (i, 0)),
    out_shape=jax.ShapeDtypeStruct((N_PAD, D), jnp.float32),
)


def _fin_body(a_ref, yp_ref, dinv_ref, b_ref, o_ref):
    t = a_ref[0] + a_ref[1] - yp_ref[...]
    o_ref[...] = t * dinv_ref[...] + b_ref[...]


_fin_call = pl.pallas_call(
    _fin_body,
    grid=(GRID,),
    in_specs=[
        pl.BlockSpec((NC, BR, D), lambda i: (0, i, 0)),
        pl.BlockSpec((BR, D), lambda i: (i, 0)),
        pl.BlockSpec((BR, 1), lambda i: (i, 0)),
        pl.BlockSpec((1, D), lambda i: (0, 0)),
    ],
    out_specs=pl.BlockSpec((BR, D), lambda i: (i, 0)),
    out_shape=jax.ShapeDtypeStruct((N_PAD, D), jnp.float32),
)


def kernel(x, edge_index, W1, b1, W2, b2, W3, b3):
    src = edge_index[0].astype(jnp.int32)
    dst = edge_index[1].astype(jnp.int32)
    pad = jnp.full((E_PAD - E,), N, jnp.int32)
    srcf = jnp.concatenate([src, pad])
    dstf = jnp.concatenate([dst, pad])

    def _tile_block(nchunks, off):
        n = NS * nchunks * CHUNK
        blk = jnp.stack(
            [srcf[off:off + n].reshape(NS, nchunks, 1, CHUNK),
             dstf[off:off + n].reshape(NS, nchunks, 1, CHUNK)], axis=2)
        fill = jnp.full((NS, ECP - nchunks, 2, 1, CHUNK), N, jnp.int32)
        return jnp.concatenate([blk, fill], axis=1)

    fast_blk = _tile_block(ECF, 0)
    slow_blk = _tile_block(ECS, NS * ECF * CHUNK)
    eip = jnp.concatenate(
        [fast_blk, slow_blk] if FAST_CID == 0 else [slow_blk, fast_blk],
        axis=0)
    xp = jnp.pad(x, ((0, N_PAD - N), (0, 0)))
    zeros = jnp.zeros((N_PAD,), jnp.float32)

    degp = _deg_kernel(eip, zeros)
    dinv = _prep_call(degp.reshape(NC, N_PAD // 128, 128)).reshape(N_PAD, 1)

    b1r, b2r, b3r = (b.reshape(1, D) for b in (b1, b2, b3))
    y1 = _y1_call(xp, dinv, W1)
    a1 = _agg_kernel(y1, eip)
    y2 = _mid_call(a1, y1, dinv, b1r, W2)
    a2 = _agg_kernel(y2, eip)
    y3 = _mid_call(a2, y2, dinv, b2r, W3)
    a3 = _agg_kernel(y3, eip)
    out = _fin_call(a3, y3, dinv, b3r)
    return out[:N]


# speedup vs baseline: 1.0391x; 1.0010x over previous
import functools

import jax
import jax.numpy as jnp
from jax import lax
from jax.experimental import pallas as pl
from jax.experimental.pallas import tpu as pltpu
from jax.experimental.pallas import tpu_sc as plsc

N = 10000
D = 128
E = 320000
NC, NS = 2, 16
NW = NC * NS
N_PAD = 10240
RPT = N_PAD // NS
CHUNK = 256
FAST_CID = 1
ECF = 40
ECS = 40
ECP = ECF + 4
E_PAD = NS * (ECF + ECS) * CHUNK

_mesh = plsc.VectorSubcoreMesh(core_axis_name="c", subcore_axis_name="s")


@functools.partial(
    pl.kernel,
    mesh=_mesh,
    out_type=jax.ShapeDtypeStruct((NC, N_PAD), jnp.float32),
    scratch_types=[
        pltpu.VMEM((ECP, 2, 1, CHUNK), jnp.int32),
        pltpu.VMEM((1, CHUNK), jnp.float32),
        pltpu.VMEM_SHARED((N_PAD,), jnp.float32),
    ],
)
def _deg_kernel(ei_hbm, zeros_hbm, out_hbm, idx_v, ones_v, acc):
    cid = lax.axis_index("c")
    sid = lax.axis_index("s")
    wid = cid * NS + sid
    for i in range(CHUNK // 16):
        ones_v[0, pl.ds(i * 16, 16)] = jnp.ones((16,), jnp.float32)
    pltpu.sync_copy(zeros_hbm.at[pl.ds(sid * RPT, RPT)],
                    acc.at[pl.ds(sid * RPT, RPT)])
    pltpu.sync_copy(ei_hbm.at[wid], idx_v)
    plsc.subcore_barrier()

    def body(j, c):
        pltpu.sync_copy(ones_v.at[0], acc.at[idx_v.at[j, 1, 0]], add=True)
        return c

    lax.fori_loop(0, jnp.where(cid == FAST_CID, ECF, ECS), body, 0)
    plsc.subcore_barrier()
    pltpu.sync_copy(acc.at[pl.ds(sid * RPT, RPT)],
                    out_hbm.at[cid, pl.ds(sid * RPT, RPT)])


@functools.partial(
    pl.kernel,
    mesh=_mesh,
    out_type=jax.ShapeDtypeStruct((NC, N_PAD, D), jnp.float32),
    scratch_types=[
        pltpu.VMEM((4, 2, 1, CHUNK), jnp.int32),
        pltpu.VMEM((CHUNK, D), jnp.float32),
        pltpu.VMEM_SHARED((N_PAD, D), jnp.float32),
        pltpu.SemaphoreType.DMA,
        pltpu.SemaphoreType.DMA,
        pltpu.SemaphoreType.DMA,
        pltpu.SemaphoreType.DMA,
        pltpu.SemaphoreType.DMA,
    ],
)
def _agg_kernel(y_hbm, ei_hbm, out_hbm, ring, rows_v, acc,
                isem0, isem1, isem2, isem3, gsem):
    cid = lax.axis_index("c")
    sid = lax.axis_index("s")
    wid = cid * NS + sid
    isems = (isem0, isem1, isem2, isem3)
    pltpu.sync_copy(y_hbm.at[pl.ds(sid * RPT, RPT)],
                    acc.at[pl.ds(sid * RPT, RPT)])
    plsc.subcore_barrier()

    for s in range(4):
        pltpu.async_copy(ei_hbm.at[wid, s], ring.at[s], isems[s])

    def body(i, c):
        for b in range(4):
            j = 4 * i + b
            pltpu.make_async_copy(ei_hbm.at[wid, j], ring.at[b],
                                  isems[b]).wait()
            pltpu.async_copy(y_hbm.at[ring.at[b, 0, 0]], rows_v, gsem).wait()
            pltpu.sync_copy(rows_v, acc.at[ring.at[b, 1, 0]], add=True)
            pltpu.async_copy(ei_hbm.at[wid, j + 4], ring.at[b], isems[b])
        return c

    lax.fori_loop(0, jnp.where(cid == FAST_CID, ECF // 4, ECS // 4), body, 0)
    for s in range(4):
        pltpu.make_async_copy(ei_hbm.at[wid, 0], ring.at[s], isems[s]).wait()
    plsc.subcore_barrier()
    pltpu.sync_copy(acc.at[pl.ds(sid * RPT, RPT)],
                    out_hbm.at[cid, pl.ds(sid * RPT, RPT)])


def _prep_body(degp_ref, o_ref):
    deg = degp_ref[0] + degp_ref[1] + 1.0
    dv = lax.rsqrt(deg)
    row = lax.broadcasted_iota(jnp.int32, (N_PAD // 128, 128), 0)
    col = lax.broadcasted_iota(jnp.int32, (N_PAD // 128, 128), 1)
    o_ref[...] = jnp.where(row * 128 + col < N, dv, 0.0)


_prep_call = pl.pallas_call(
    _prep_body,
    out_shape=jax.ShapeDtypeStruct((N_PAD // 128, 128), jnp.float32),
)

BR = 256
GRID = N_PAD // BR


def _y1_body(x_ref, dinv_ref, w_ref, o_ref):
    o_ref[...] = jnp.dot(x_ref[...] * dinv_ref[...], w_ref[...],
                         preferred_element_type=jnp.float32)


_y1_call = pl.pallas_call(
    _y1_body,
    grid=(GRID,),
    in_specs=[
        pl.BlockSpec((BR, D), lambda i: (i, 0)),
        pl.BlockSpec((BR, 1), lambda i: (i, 0)),
        pl.BlockSpec((D, D), lambda i: (0, 0)),
    ],
    out_specs=pl.BlockSpec((BR, D), lambda i: (i, 0)),
    out_shape=jax.ShapeDtypeStruct((N_PAD, D), jnp.float32),
)


def _mid_body(a_ref, yp_ref, dinv_ref, b_ref, w_ref, o_ref):
    t = a_ref[0] + a_ref[1] - yp_ref[...]
    t = t * dinv_ref[...] + b_ref[...]
    t = jnp.maximum(t, 0.0)
    o_ref[...] = jnp.dot(t * dinv_ref[...], w_ref[...],
                         preferred_element_type=jnp.float32)


_mid_call = pl.pallas_call(
    _mid_body,
    grid=(GRID,),
    in_specs=[
        pl.BlockSpec((NC, BR, D), lambda i: (0, i, 0)),
        pl.BlockSpec((BR, D), lambda i: (i, 0)),
        pl.BlockSpec((BR, 1), lambda i: (i, 0)),
        pl.BlockSpec((1, D), lambda i: (0, 0)),
        pl.BlockSpec((D, D), lambda i: (0, 0)),
    ],
    out_specs=pl.BlockSpec((BR, D), lambda i: (i, 0)),
    out_shape=jax.ShapeDtypeStruct((N_PAD, D), jnp.float32),
)


def _fin_body(a_ref, yp_ref, dinv_ref, b_ref, o_ref):
    t = a_ref[0] + a_ref[1] - yp_ref[...]
    o_ref[...] = t * dinv_ref[...] + b_ref[...]


_fin_call = pl.pallas_call(
    _fin_body,
    grid=(GRID,),
    in_specs=[
        pl.BlockSpec((NC, BR, D), lambda i: (0, i, 0)),
        pl.BlockSpec((BR, D), lambda i: (i, 0)),
        pl.BlockSpec((BR, 1), lambda i: (i, 0)),
        pl.BlockSpec((1, D), lambda i: (0, 0)),
    ],
    out_specs=pl.BlockSpec((BR, D), lambda i: (i, 0)),
    out_shape=jax.ShapeDtypeStruct((N_PAD, D), jnp.float32),
)


def kernel(x, edge_index, W1, b1, W2, b2, W3, b3):
    src = edge_index[0].astype(jnp.int32)
    dst = edge_index[1].astype(jnp.int32)
    pad = jnp.full((E_PAD - E,), N, jnp.int32)
    srcf = jnp.concatenate([src, pad])
    dstf = jnp.concatenate([dst, pad])

    def _tile_block(nchunks, off):
        n = NS * nchunks * CHUNK
        blk = jnp.stack(
            [srcf[off:off + n].reshape(NS, nchunks, 1, CHUNK),
             dstf[off:off + n].reshape(NS, nchunks, 1, CHUNK)], axis=2)
        fill = jnp.full((NS, ECP - nchunks, 2, 1, CHUNK), N, jnp.int32)
        return jnp.concatenate([blk, fill], axis=1)

    fast_blk = _tile_block(ECF, 0)
    slow_blk = _tile_block(ECS, NS * ECF * CHUNK)
    eip = jnp.concatenate(
        [fast_blk, slow_blk] if FAST_CID == 0 else [slow_blk, fast_blk],
        axis=0)
    xp = jnp.pad(x, ((0, N_PAD - N), (0, 0)))
    zeros = jnp.zeros((N_PAD,), jnp.float32)

    degp = _deg_kernel(eip, zeros)
    dinv = _prep_call(degp.reshape(NC, N_PAD // 128, 128)).reshape(N_PAD, 1)

    b1r, b2r, b3r = (b.reshape(1, D) for b in (b1, b2, b3))
    y1 = _y1_call(xp, dinv, W1)
    a1 = _agg_kernel(y1, eip)
    y2 = _mid_call(a1, y1, dinv, b1r, W2)
    a2 = _agg_kernel(y2, eip)
    y3 = _mid_call(a2, y2, dinv, b2r, W3)
    a3 = _agg_kernel(y3, eip)
    out = _fin_call(a3, y3, dinv, b3r)
    return out[:N]


# speedup vs baseline: 1.0394x; 1.0003x over previous
import functools

import jax
import jax.numpy as jnp
from jax import lax
from jax.experimental import pallas as pl
from jax.experimental.pallas import tpu as pltpu
from jax.experimental.pallas import tpu_sc as plsc

N = 10000
D = 128
E = 320000
NC, NS = 2, 16
NW = NC * NS
N_PAD = 10240
RPT = N_PAD // NS
CHUNK = 256
FAST_CID = 1
ECF = 40
ECS = 40
ECP = ECF + 4
E_PAD = NS * (ECF + ECS) * CHUNK

_mesh = plsc.VectorSubcoreMesh(core_axis_name="c", subcore_axis_name="s")


@functools.partial(
    pl.kernel,
    mesh=_mesh,
    out_type=jax.ShapeDtypeStruct((NC, N_PAD), jnp.float32),
    scratch_types=[
        pltpu.VMEM((ECP, 2, 1, CHUNK), jnp.int32),
        pltpu.VMEM((1, CHUNK), jnp.float32),
        pltpu.VMEM_SHARED((N_PAD,), jnp.float32),
    ],
)
def _deg_kernel(ei_hbm, zeros_hbm, out_hbm, idx_v, ones_v, acc):
    cid = lax.axis_index("c")
    sid = lax.axis_index("s")
    wid = cid * NS + sid
    for i in range(CHUNK // 16):
        ones_v[0, pl.ds(i * 16, 16)] = jnp.ones((16,), jnp.float32)
    pltpu.sync_copy(zeros_hbm.at[pl.ds(sid * RPT, RPT)],
                    acc.at[pl.ds(sid * RPT, RPT)])
    pltpu.sync_copy(ei_hbm.at[wid], idx_v)
    plsc.subcore_barrier()

    def body(j, c):
        pltpu.sync_copy(ones_v.at[0], acc.at[idx_v.at[j, 1, 0]], add=True)
        return c

    lax.fori_loop(0, ECF, body, 0)
    plsc.subcore_barrier()
    pltpu.sync_copy(acc.at[pl.ds(sid * RPT, RPT)],
                    out_hbm.at[cid, pl.ds(sid * RPT, RPT)])


@functools.partial(
    pl.kernel,
    mesh=_mesh,
    out_type=jax.ShapeDtypeStruct((NC, N_PAD, D), jnp.float32),
    scratch_types=[
        pltpu.VMEM((4, 2, 1, CHUNK), jnp.int32),
        pltpu.VMEM((CHUNK, D), jnp.float32),
        pltpu.VMEM_SHARED((N_PAD, D), jnp.float32),
        pltpu.SemaphoreType.DMA,
        pltpu.SemaphoreType.DMA,
        pltpu.SemaphoreType.DMA,
        pltpu.SemaphoreType.DMA,
        pltpu.SemaphoreType.DMA,
    ],
)
def _agg_kernel(y_hbm, ei_hbm, out_hbm, ring, rows_v, acc,
                isem0, isem1, isem2, isem3, gsem):
    cid = lax.axis_index("c")
    sid = lax.axis_index("s")
    wid = cid * NS + sid
    isems = (isem0, isem1, isem2, isem3)
    pltpu.sync_copy(y_hbm.at[pl.ds(sid * RPT, RPT)],
                    acc.at[pl.ds(sid * RPT, RPT)])
    plsc.subcore_barrier()

    for s in range(4):
        pltpu.async_copy(ei_hbm.at[wid, s], ring.at[s], isems[s])

    def body(i, c):
        for b in range(4):
            j = 4 * i + b
            pltpu.make_async_copy(ei_hbm.at[wid, j], ring.at[b],
                                  isems[b]).wait()
            pltpu.async_copy(y_hbm.at[ring.at[b, 0, 0]], rows_v, gsem).wait()
            pltpu.sync_copy(rows_v, acc.at[ring.at[b, 1, 0]], add=True)
            pltpu.async_copy(ei_hbm.at[wid, j + 4], ring.at[b], isems[b])
        return c

    lax.fori_loop(0, ECF // 4, body, 0)
    for s in range(4):
        pltpu.make_async_copy(ei_hbm.at[wid, 0], ring.at[s], isems[s]).wait()
    plsc.subcore_barrier()
    pltpu.sync_copy(acc.at[pl.ds(sid * RPT, RPT)],
                    out_hbm.at[cid, pl.ds(sid * RPT, RPT)])


def _prep_body(degp_ref, o_ref):
    deg = degp_ref[0] + degp_ref[1] + 1.0
    dv = lax.rsqrt(deg)
    row = lax.broadcasted_iota(jnp.int32, (N_PAD // 128, 128), 0)
    col = lax.broadcasted_iota(jnp.int32, (N_PAD // 128, 128), 1)
    o_ref[...] = jnp.where(row * 128 + col < N, dv, 0.0)


_prep_call = pl.pallas_call(
    _prep_body,
    out_shape=jax.ShapeDtypeStruct((N_PAD // 128, 128), jnp.float32),
)

BR = 256
GRID = N_PAD // BR


def _y1_body(x_ref, dinv_ref, w_ref, o_ref):
    o_ref[...] = jnp.dot(x_ref[...] * dinv_ref[...], w_ref[...],
                         preferred_element_type=jnp.float32)


_y1_call = pl.pallas_call(
    _y1_body,
    grid=(GRID,),
    in_specs=[
        pl.BlockSpec((BR, D), lambda i: (i, 0)),
        pl.BlockSpec((BR, 1), lambda i: (i, 0)),
        pl.BlockSpec((D, D), lambda i: (0, 0)),
    ],
    out_specs=pl.BlockSpec((BR, D), lambda i: (i, 0)),
    out_shape=jax.ShapeDtypeStruct((N_PAD, D), jnp.float32),
)


def _mid_body(a_ref, yp_ref, dinv_ref, b_ref, w_ref, o_ref):
    t = a_ref[0] + a_ref[1] - yp_ref[...]
    t = t * dinv_ref[...] + b_ref[...]
    t = jnp.maximum(t, 0.0)
    o_ref[...] = jnp.dot(t * dinv_ref[...], w_ref[...],
                         preferred_element_type=jnp.float32)


_mid_call = pl.pallas_call(
    _mid_body,
    grid=(GRID,),
    in_specs=[
        pl.BlockSpec((NC, BR, D), lambda i: (0, i, 0)),
        pl.BlockSpec((BR, D), lambda i: (i, 0)),
        pl.BlockSpec((BR, 1), lambda i: (i, 0)),
        pl.BlockSpec((1, D), lambda i: (0, 0)),
        pl.BlockSpec((D, D), lambda i: (0, 0)),
    ],
    out_specs=pl.BlockSpec((BR, D), lambda i: (i, 0)),
    out_shape=jax.ShapeDtypeStruct((N_PAD, D), jnp.float32),
)


def _fin_body(a_ref, yp_ref, dinv_ref, b_ref, o_ref):
    t = a_ref[0] + a_ref[1] - yp_ref[...]
    o_ref[...] = t * dinv_ref[...] + b_ref[...]


_fin_call = pl.pallas_call(
    _fin_body,
    grid=(GRID,),
    in_specs=[
        pl.BlockSpec((NC, BR, D), lambda i: (0, i, 0)),
        pl.BlockSpec((BR, D), lambda i: (i, 0)),
        pl.BlockSpec((BR, 1), lambda i: (i, 0)),
        pl.BlockSpec((1, D), lambda i: (0, 0)),
    ],
    out_specs=pl.BlockSpec((BR, D), lambda i: (i, 0)),
    out_shape=jax.ShapeDtypeStruct((N_PAD, D), jnp.float32),
)


def kernel(x, edge_index, W1, b1, W2, b2, W3, b3):
    src = edge_index[0].astype(jnp.int32)
    dst = edge_index[1].astype(jnp.int32)
    pad = jnp.full((E_PAD - E,), N, jnp.int32)
    srcf = jnp.concatenate([src, pad])
    dstf = jnp.concatenate([dst, pad])

    def _tile_block(nchunks, off):
        n = NS * nchunks * CHUNK
        blk = jnp.stack(
            [srcf[off:off + n].reshape(NS, nchunks, 1, CHUNK),
             dstf[off:off + n].reshape(NS, nchunks, 1, CHUNK)], axis=2)
        fill = jnp.full((NS, ECP - nchunks, 2, 1, CHUNK), N, jnp.int32)
        return jnp.concatenate([blk, fill], axis=1)

    fast_blk = _tile_block(ECF, 0)
    slow_blk = _tile_block(ECS, NS * ECF * CHUNK)
    eip = jnp.concatenate(
        [fast_blk, slow_blk] if FAST_CID == 0 else [slow_blk, fast_blk],
        axis=0)
    xp = jnp.pad(x, ((0, N_PAD - N), (0, 0)))
    zeros = jnp.zeros((N_PAD,), jnp.float32)

    degp = _deg_kernel(eip, zeros)
    dinv = _prep_call(degp.reshape(NC, N_PAD // 128, 128)).reshape(N_PAD, 1)

    b1r, b2r, b3r = (b.reshape(1, D) for b in (b1, b2, b3))
    y1 = _y1_call(xp, dinv, W1)
    a1 = _agg_kernel(y1, eip)
    y2 = _mid_call(a1, y1, dinv, b1r, W2)
    a2 = _agg_kernel(y2, eip)
    y3 = _mid_call(a2, y2, dinv, b2r, W3)
    a3 = _agg_kernel(y3, eip)
    out = _fin_call(a3, y3, dinv, b3r)
    return out[:N]


# speedup vs baseline: 1.1355x; 1.0924x over previous
import functools

import jax
import jax.numpy as jnp
from jax import lax
from jax.experimental import pallas as pl
from jax.experimental.pallas import tpu as pltpu
from jax.experimental.pallas import tpu_sc as plsc

N = 10000
D = 128
E = 320000
NC, NS = 2, 16
NW = NC * NS
N_PAD = 10240
RPT = N_PAD // NS
CHUNK = 256
EC = 40
ECP = EC + 4
E_PAD = NW * EC * CHUNK

_mesh = plsc.VectorSubcoreMesh(core_axis_name="c", subcore_axis_name="s")


@functools.partial(
    pl.kernel,
    mesh=_mesh,
    out_type=jax.ShapeDtypeStruct((NC, N_PAD), jnp.float32),
    scratch_types=[
        pltpu.VMEM((ECP, 2, 1, CHUNK), jnp.int32),
        pltpu.VMEM((1, CHUNK), jnp.float32),
        pltpu.VMEM_SHARED((N_PAD,), jnp.float32),
    ],
)
def _deg_kernel(ei_hbm, zeros_hbm, out_hbm, idx_v, ones_v, acc):
    cid = lax.axis_index("c")
    sid = lax.axis_index("s")
    wid = cid * NS + sid
    for i in range(CHUNK // 16):
        ones_v[0, pl.ds(i * 16, 16)] = jnp.ones((16,), jnp.float32)
    pltpu.sync_copy(zeros_hbm.at[pl.ds(sid * RPT, RPT)],
                    acc.at[pl.ds(sid * RPT, RPT)])
    pltpu.sync_copy(ei_hbm.at[wid], idx_v)
    plsc.subcore_barrier()

    def body(j, c):
        pltpu.sync_copy(ones_v.at[0], acc.at[idx_v.at[j, 1, 0]], add=True)
        return c

    lax.fori_loop(0, EC, body, 0)
    plsc.subcore_barrier()
    pltpu.sync_copy(acc.at[pl.ds(sid * RPT, RPT)],
                    out_hbm.at[cid, pl.ds(sid * RPT, RPT)])


@functools.partial(
    pl.kernel,
    mesh=_mesh,
    out_type=jax.ShapeDtypeStruct((NC, N_PAD, D), jnp.float32),
    scratch_types=[
        pltpu.VMEM((4, 2, 1, CHUNK), jnp.int32),
        pltpu.VMEM((CHUNK, D), jnp.float32),
        pltpu.VMEM_SHARED((N_PAD, D), jnp.float32),
        pltpu.SemaphoreType.DMA,
        pltpu.SemaphoreType.DMA,
        pltpu.SemaphoreType.DMA,
        pltpu.SemaphoreType.DMA,
        pltpu.SemaphoreType.DMA,
    ],
)
def _agg_kernel(y_hbm, ei_hbm, out_hbm, ring, rows_v, acc,
                isem0, isem1, isem2, isem3, gsem):
    cid = lax.axis_index("c")
    sid = lax.axis_index("s")
    wid = cid * NS + sid
    isems = (isem0, isem1, isem2, isem3)
    pltpu.sync_copy(y_hbm.at[pl.ds(sid * RPT, RPT)],
                    acc.at[pl.ds(sid * RPT, RPT)])
    plsc.subcore_barrier()

    for s in range(4):
        pltpu.async_copy(ei_hbm.at[wid, s], ring.at[s], isems[s])

    def body(i, c):
        for b in range(4):
            j = 4 * i + b
            pltpu.make_async_copy(ei_hbm.at[wid, j], ring.at[b],
                                  isems[b]).wait()
            pltpu.async_copy(y_hbm.at[ring.at[b, 0, 0]], rows_v, gsem).wait()
            pltpu.sync_copy(rows_v, acc.at[ring.at[b, 1, 0]], add=True)
            pltpu.async_copy(ei_hbm.at[wid, j + 4], ring.at[b], isems[b])
        return c

    lax.fori_loop(0, EC // 4, body, 0)
    for s in range(4):
        pltpu.make_async_copy(ei_hbm.at[wid, 0], ring.at[s], isems[s]).wait()
    plsc.subcore_barrier()
    pltpu.sync_copy(acc.at[pl.ds(sid * RPT, RPT)],
                    out_hbm.at[cid, pl.ds(sid * RPT, RPT)])


def _prep_body(degp_ref, o_ref):
    deg = degp_ref[0] + degp_ref[1] + 1.0
    dv = lax.rsqrt(deg)
    row = lax.broadcasted_iota(jnp.int32, (N_PAD // 128, 128), 0)
    col = lax.broadcasted_iota(jnp.int32, (N_PAD // 128, 128), 1)
    o_ref[...] = jnp.where(row * 128 + col < N, dv, 0.0)


_prep_call = pl.pallas_call(
    _prep_body,
    out_shape=jax.ShapeDtypeStruct((N_PAD // 128, 128), jnp.float32),
)

BR = 256
GRID = N_PAD // BR


def _y1_body(x_ref, dinv_ref, w_ref, o_ref):
    o_ref[...] = jnp.dot(x_ref[...] * dinv_ref[...], w_ref[...],
                         preferred_element_type=jnp.float32)


_y1_call = pl.pallas_call(
    _y1_body,
    grid=(GRID,),
    in_specs=[
        pl.BlockSpec((BR, D), lambda i: (i, 0)),
        pl.BlockSpec((BR, 1), lambda i: (i, 0)),
        pl.BlockSpec((D, D), lambda i: (0, 0)),
    ],
    out_specs=pl.BlockSpec((BR, D), lambda i: (i, 0)),
    out_shape=jax.ShapeDtypeStruct((N_PAD, D), jnp.float32),
)


def _mid_body(a_ref, yp_ref, dinv_ref, b_ref, w_ref, o_ref):
    t = a_ref[0] + a_ref[1] - yp_ref[...]
    t = t * dinv_ref[...] + b_ref[...]
    t = jnp.maximum(t, 0.0)
    o_ref[...] = jnp.dot(t * dinv_ref[...], w_ref[...],
                         preferred_element_type=jnp.float32)


_mid_call = pl.pallas_call(
    _mid_body,
    grid=(GRID,),
    in_specs=[
        pl.BlockSpec((NC, BR, D), lambda i: (0, i, 0)),
        pl.BlockSpec((BR, D), lambda i: (i, 0)),
        pl.BlockSpec((BR, 1), lambda i: (i, 0)),
        pl.BlockSpec((1, D), lambda i: (0, 0)),
        pl.BlockSpec((D, D), lambda i: (0, 0)),
    ],
    out_specs=pl.BlockSpec((BR, D), lambda i: (i, 0)),
    out_shape=jax.ShapeDtypeStruct((N_PAD, D), jnp.float32),
)


def _fin_body(a_ref, yp_ref, dinv_ref, b_ref, o_ref):
    t = a_ref[0] + a_ref[1] - yp_ref[...]
    o_ref[...] = t * dinv_ref[...] + b_ref[...]


_fin_call = pl.pallas_call(
    _fin_body,
    grid=(GRID,),
    in_specs=[
        pl.BlockSpec((NC, BR, D), lambda i: (0, i, 0)),
        pl.BlockSpec((BR, D), lambda i: (i, 0)),
        pl.BlockSpec((BR, 1), lambda i: (i, 0)),
        pl.BlockSpec((1, D), lambda i: (0, 0)),
    ],
    out_specs=pl.BlockSpec((BR, D), lambda i: (i, 0)),
    out_shape=jax.ShapeDtypeStruct((N_PAD, D), jnp.float32),
)


def kernel(x, edge_index, W1, b1, W2, b2, W3, b3):
    src = edge_index[0].astype(jnp.int32)
    dst = edge_index[1].astype(jnp.int32)
    pad = jnp.full((E_PAD - E,), N, jnp.int32)
    src_all = jnp.concatenate([src, pad]).reshape(NW, EC, 1, CHUNK)
    dst_all = jnp.concatenate([dst, pad]).reshape(NW, EC, 1, CHUNK)
    padc = jnp.full((NW, ECP - EC, 2, 1, CHUNK), N, jnp.int32)
    eip = jnp.concatenate(
        [jnp.stack([src_all, dst_all], axis=2), padc], axis=1)
    xp = jnp.pad(x, ((0, N_PAD - N), (0, 0)))
    zeros = jnp.zeros((N_PAD,), jnp.float32)

    degp = _deg_kernel(eip, zeros)
    dinv = _prep_call(degp.reshape(NC, N_PAD // 128, 128)).reshape(N_PAD, 1)

    b1r, b2r, b3r = (b.reshape(1, D) for b in (b1, b2, b3))
    y1 = _y1_call(xp, dinv, W1)
    a1 = _agg_kernel(y1, eip)
    y2 = _mid_call(a1, y1, dinv, b1r, W2)
    a2 = _agg_kernel(y2, eip)
    y3 = _mid_call(a2, y2, dinv, b2r, W3)
    a3 = _agg_kernel(y3, eip)
    out = _fin_call(a3, y3, dinv, b3r)
    return out[:N]


# speedup vs baseline: 2.8272x; 2.4899x over previous
import functools

import jax
import jax.numpy as jnp
from jax import lax
from jax.experimental import pallas as pl
from jax.experimental.pallas import tpu as pltpu
from jax.experimental.pallas import tpu_sc as plsc

N = 10000
D = 128
E = 320000
NC, NS = 2, 16
NW = NC * NS
N_PAD = 10240
RPT = N_PAD // NS
CHUNK = 256
EC = 40
ECP = EC + 4
E_PAD = NW * EC * CHUNK

_mesh = plsc.VectorSubcoreMesh(core_axis_name="c", subcore_axis_name="s")


@functools.partial(
    pl.kernel,
    mesh=_mesh,
    out_type=jax.ShapeDtypeStruct((NC, N_PAD), jnp.float32),
    scratch_types=[
        pltpu.VMEM((ECP, 2, 1, CHUNK), jnp.int32),
        pltpu.VMEM((1, CHUNK), jnp.float32),
        pltpu.VMEM_SHARED((N_PAD,), jnp.float32),
    ],
)
def _deg_kernel(ei_hbm, zeros_hbm, out_hbm, idx_v, ones_v, acc):
    cid = lax.axis_index("c")
    sid = lax.axis_index("s")
    wid = cid * NS + sid
    for i in range(CHUNK // 16):
        ones_v[0, pl.ds(i * 16, 16)] = jnp.ones((16,), jnp.float32)
    pltpu.sync_copy(zeros_hbm.at[pl.ds(sid * RPT, RPT)],
                    acc.at[pl.ds(sid * RPT, RPT)])
    pltpu.sync_copy(ei_hbm.at[wid], idx_v)
    plsc.subcore_barrier()

    def body(j, c):
        pltpu.sync_copy(ones_v.at[0], acc.at[idx_v.at[j, 1, 0]], add=True)
        return c

    lax.fori_loop(0, EC, body, 0)
    plsc.subcore_barrier()
    pltpu.sync_copy(acc.at[pl.ds(sid * RPT, RPT)],
                    out_hbm.at[cid, pl.ds(sid * RPT, RPT)])


@functools.partial(
    pl.kernel,
    mesh=_mesh,
    out_type=jax.ShapeDtypeStruct((NC, N_PAD, D), jnp.float32),
    scratch_types=[
        pltpu.VMEM((4, 2, 1, CHUNK), jnp.int32),
        pltpu.VMEM((CHUNK, D), jnp.float32),
        pltpu.VMEM_SHARED((N_PAD, D), jnp.float32),
        pltpu.SemaphoreType.DMA,
        pltpu.SemaphoreType.DMA,
        pltpu.SemaphoreType.DMA,
        pltpu.SemaphoreType.DMA,
        pltpu.SemaphoreType.DMA,
    ],
)
def _agg_kernel(y_hbm, ei_hbm, out_hbm, ring, rows_v, acc,
                isem0, isem1, isem2, isem3, gsem):
    cid = lax.axis_index("c")
    sid = lax.axis_index("s")
    wid = cid * NS + sid
    isems = (isem0, isem1, isem2, isem3)
    pltpu.sync_copy(y_hbm.at[pl.ds(sid * RPT, RPT)],
                    acc.at[pl.ds(sid * RPT, RPT)])
    plsc.subcore_barrier()

    for s in range(4):
        pltpu.async_copy(ei_hbm.at[wid, s], ring.at[s], isems[s])

    def body(i, c):
        for b in range(4):
            j = 4 * i + b
            pltpu.make_async_copy(ei_hbm.at[wid, j], ring.at[b],
                                  isems[b]).wait()
            pltpu.async_copy(y_hbm.at[ring.at[b, 0, 0]], rows_v, gsem).wait()
            pltpu.sync_copy(rows_v, acc.at[ring.at[b, 1, 0]], add=True)
            pltpu.async_copy(ei_hbm.at[wid, j + 4], ring.at[b], isems[b])
        return c

    lax.fori_loop(0, EC // 4, body, 0)
    for s in range(4):
        pltpu.make_async_copy(ei_hbm.at[wid, 0], ring.at[s], isems[s]).wait()
    plsc.subcore_barrier()
    pltpu.sync_copy(acc.at[pl.ds(sid * RPT, RPT)],
                    out_hbm.at[cid, pl.ds(sid * RPT, RPT)])


def _prep_body(degp_ref, o_ref):
    deg = degp_ref[0] + degp_ref[1] + 1.0
    dv = lax.rsqrt(deg)
    row = lax.broadcasted_iota(jnp.int32, (N_PAD // 128, 128), 0)
    col = lax.broadcasted_iota(jnp.int32, (N_PAD // 128, 128), 1)
    o_ref[...] = jnp.where(row * 128 + col < N, dv, 0.0)


_prep_call = pl.pallas_call(
    _prep_body,
    out_shape=jax.ShapeDtypeStruct((N_PAD // 128, 128), jnp.float32),
)

BR = 256
GRID = N_PAD // BR


def _y1_body(x_ref, dinv_ref, w_ref, o_ref):
    o_ref[...] = jnp.dot(x_ref[...] * dinv_ref[...], w_ref[...],
                         preferred_element_type=jnp.float32)


_y1_call = pl.pallas_call(
    _y1_body,
    grid=(GRID,),
    in_specs=[
        pl.BlockSpec((BR, D), lambda i: (i, 0)),
        pl.BlockSpec((BR, 1), lambda i: (i, 0)),
        pl.BlockSpec((D, D), lambda i: (0, 0)),
    ],
    out_specs=pl.BlockSpec((BR, D), lambda i: (i, 0)),
    out_shape=jax.ShapeDtypeStruct((N_PAD, D), jnp.float32),
)


def _mid_body(a_ref, yp_ref, dinv_ref, b_ref, w_ref, o_ref):
    t = a_ref[0] + a_ref[1] - yp_ref[...]
    t = t * dinv_ref[...] + b_ref[...]
    t = jnp.maximum(t, 0.0)
    o_ref[...] = jnp.dot(t * dinv_ref[...], w_ref[...],
                         preferred_element_type=jnp.float32)


_mid_call = pl.pallas_call(
    _mid_body,
    grid=(GRID,),
    in_specs=[
        pl.BlockSpec((NC, BR, D), lambda i: (0, i, 0)),
        pl.BlockSpec((BR, D), lambda i: (i, 0)),
        pl.BlockSpec((BR, 1), lambda i: (i, 0)),
        pl.BlockSpec((1, D), lambda i: (0, 0)),
        pl.BlockSpec((D, D), lambda i: (0, 0)),
    ],
    out_specs=pl.BlockSpec((BR, D), lambda i: (i, 0)),
    out_shape=jax.ShapeDtypeStruct((N_PAD, D), jnp.float32),
)


def _fin_body(a_ref, yp_ref, dinv_ref, b_ref, o_ref):
    t = a_ref[0] + a_ref[1] - yp_ref[...]
    o_ref[...] = t * dinv_ref[...] + b_ref[...]


_fin_call = pl.pallas_call(
    _fin_body,
    grid=(GRID,),
    in_specs=[
        pl.BlockSpec((NC, BR, D), lambda i: (0, i, 0)),
        pl.BlockSpec((BR, D), lambda i: (i, 0)),
        pl.BlockSpec((BR, 1), lambda i: (i, 0)),
        pl.BlockSpec((1, D), lambda i: (0, 0)),
    ],
    out_specs=pl.BlockSpec((BR, D), lambda i: (i, 0)),
    out_shape=jax.ShapeDtypeStruct((N_PAD, D), jnp.float32),
)


def kernel(x, edge_index, W1, b1, W2, b2, W3, b3):
    src = edge_index[0].astype(jnp.int32)
    dst = edge_index[1].astype(jnp.int32)
    npad = E_PAD - E
    pad_src = jnp.arange(npad, dtype=jnp.int32) % N_PAD
    pad_dst = N + jnp.arange(npad, dtype=jnp.int32) % (N_PAD - N)
    src_all = jnp.concatenate([src, pad_src]).reshape(NW, EC, 1, CHUNK)
    dst_all = jnp.concatenate([dst, pad_dst]).reshape(NW, EC, 1, CHUNK)
    ncp = NW * (ECP - EC) * CHUNK
    padc = jnp.stack(
        [(jnp.arange(ncp, dtype=jnp.int32) % N_PAD).reshape(
            NW, ECP - EC, 1, CHUNK),
         (N + jnp.arange(ncp, dtype=jnp.int32) % (N_PAD - N)).reshape(
            NW, ECP - EC, 1, CHUNK)], axis=2)
    eip = jnp.concatenate(
        [jnp.stack([src_all, dst_all], axis=2), padc], axis=1)
    xp = jnp.pad(x, ((0, N_PAD - N), (0, 0)))
    zeros = jnp.zeros((N_PAD,), jnp.float32)

    degp = _deg_kernel(eip, zeros)
    dinv = _prep_call(degp.reshape(NC, N_PAD // 128, 128)).reshape(N_PAD, 1)

    b1r, b2r, b3r = (b.reshape(1, D) for b in (b1, b2, b3))
    y1 = _y1_call(xp, dinv, W1)
    a1 = _agg_kernel(y1, eip)
    y2 = _mid_call(a1, y1, dinv, b1r, W2)
    a2 = _agg_kernel(y2, eip)
    y3 = _mid_call(a2, y2, dinv, b2r, W3)
    a3 = _agg_kernel(y3, eip)
    out = _fin_call(a3, y3, dinv, b3r)
    return out[:N]


# speedup vs baseline: 2.8684x; 1.0146x over previous
import functools

import jax
import jax.numpy as jnp
from jax import lax
from jax.experimental import pallas as pl
from jax.experimental.pallas import tpu as pltpu
from jax.experimental.pallas import tpu_sc as plsc

N = 10000
D = 128
E = 320000
NC, NS = 2, 16
NW = NC * NS
N_PAD = 10240
RPT = N_PAD // NS
CHUNK = 320
EC = 32
ECP = EC + 4
E_PAD = NW * EC * CHUNK

_mesh = plsc.VectorSubcoreMesh(core_axis_name="c", subcore_axis_name="s")


@functools.partial(
    pl.kernel,
    mesh=_mesh,
    out_type=jax.ShapeDtypeStruct((NC, N_PAD), jnp.float32),
    scratch_types=[
        pltpu.VMEM((ECP, 2, 1, CHUNK), jnp.int32),
        pltpu.VMEM((1, CHUNK), jnp.float32),
        pltpu.VMEM_SHARED((N_PAD,), jnp.float32),
    ],
)
def _deg_kernel(ei_hbm, zeros_hbm, out_hbm, idx_v, ones_v, acc):
    cid = lax.axis_index("c")
    sid = lax.axis_index("s")
    wid = cid * NS + sid
    for i in range(CHUNK // 16):
        ones_v[0, pl.ds(i * 16, 16)] = jnp.ones((16,), jnp.float32)
    pltpu.sync_copy(zeros_hbm.at[pl.ds(sid * RPT, RPT)],
                    acc.at[pl.ds(sid * RPT, RPT)])
    pltpu.sync_copy(ei_hbm.at[wid], idx_v)
    plsc.subcore_barrier()

    def body(j, c):
        pltpu.sync_copy(ones_v.at[0], acc.at[idx_v.at[j, 1, 0]], add=True)
        return c

    lax.fori_loop(0, EC, body, 0)
    plsc.subcore_barrier()
    pltpu.sync_copy(acc.at[pl.ds(sid * RPT, RPT)],
                    out_hbm.at[cid, pl.ds(sid * RPT, RPT)])


@functools.partial(
    pl.kernel,
    mesh=_mesh,
    out_type=jax.ShapeDtypeStruct((NC, N_PAD, D), jnp.float32),
    scratch_types=[
        pltpu.VMEM((4, 2, 1, CHUNK), jnp.int32),
        pltpu.VMEM((CHUNK, D), jnp.float32),
        pltpu.VMEM_SHARED((N_PAD, D), jnp.float32),
        pltpu.SemaphoreType.DMA,
        pltpu.SemaphoreType.DMA,
        pltpu.SemaphoreType.DMA,
        pltpu.SemaphoreType.DMA,
        pltpu.SemaphoreType.DMA,
    ],
)
def _agg_kernel(y_hbm, ei_hbm, out_hbm, ring, rows_v, acc,
                isem0, isem1, isem2, isem3, gsem):
    cid = lax.axis_index("c")
    sid = lax.axis_index("s")
    wid = cid * NS + sid
    isems = (isem0, isem1, isem2, isem3)
    pltpu.sync_copy(y_hbm.at[pl.ds(sid * RPT, RPT)],
                    acc.at[pl.ds(sid * RPT, RPT)])
    plsc.subcore_barrier()

    for s in range(4):
        pltpu.async_copy(ei_hbm.at[wid, s], ring.at[s], isems[s])

    def body(i, c):
        for b in range(4):
            j = 4 * i + b
            pltpu.make_async_copy(ei_hbm.at[wid, j], ring.at[b],
                                  isems[b]).wait()
            pltpu.async_copy(y_hbm.at[ring.at[b, 0, 0]], rows_v, gsem).wait()
            pltpu.sync_copy(rows_v, acc.at[ring.at[b, 1, 0]], add=True)
            pltpu.async_copy(ei_hbm.at[wid, j + 4], ring.at[b], isems[b])
        return c

    lax.fori_loop(0, EC // 4, body, 0)
    for s in range(4):
        pltpu.make_async_copy(ei_hbm.at[wid, 0], ring.at[s], isems[s]).wait()
    plsc.subcore_barrier()
    pltpu.sync_copy(acc.at[pl.ds(sid * RPT, RPT)],
                    out_hbm.at[cid, pl.ds(sid * RPT, RPT)])


def _prep_body(degp_ref, o_ref):
    deg = degp_ref[0] + degp_ref[1] + 1.0
    dv = lax.rsqrt(deg)
    row = lax.broadcasted_iota(jnp.int32, (N_PAD // 128, 128), 0)
    col = lax.broadcasted_iota(jnp.int32, (N_PAD // 128, 128), 1)
    o_ref[...] = jnp.where(row * 128 + col < N, dv, 0.0)


_prep_call = pl.pallas_call(
    _prep_body,
    out_shape=jax.ShapeDtypeStruct((N_PAD // 128, 128), jnp.float32),
)

BR = 256
GRID = N_PAD // BR


def _y1_body(x_ref, dinv_ref, w_ref, o_ref):
    o_ref[...] = jnp.dot(x_ref[...] * dinv_ref[...], w_ref[...],
                         preferred_element_type=jnp.float32)


_y1_call = pl.pallas_call(
    _y1_body,
    grid=(GRID,),
    in_specs=[
        pl.BlockSpec((BR, D), lambda i: (i, 0)),
        pl.BlockSpec((BR, 1), lambda i: (i, 0)),
        pl.BlockSpec((D, D), lambda i: (0, 0)),
    ],
    out_specs=pl.BlockSpec((BR, D), lambda i: (i, 0)),
    out_shape=jax.ShapeDtypeStruct((N_PAD, D), jnp.float32),
)


def _mid_body(a_ref, yp_ref, dinv_ref, b_ref, w_ref, o_ref):
    t = a_ref[0] + a_ref[1] - yp_ref[...]
    t = t * dinv_ref[...] + b_ref[...]
    t = jnp.maximum(t, 0.0)
    o_ref[...] = jnp.dot(t * dinv_ref[...], w_ref[...],
                         preferred_element_type=jnp.float32)


_mid_call = pl.pallas_call(
    _mid_body,
    grid=(GRID,),
    in_specs=[
        pl.BlockSpec((NC, BR, D), lambda i: (0, i, 0)),
        pl.BlockSpec((BR, D), lambda i: (i, 0)),
        pl.BlockSpec((BR, 1), lambda i: (i, 0)),
        pl.BlockSpec((1, D), lambda i: (0, 0)),
        pl.BlockSpec((D, D), lambda i: (0, 0)),
    ],
    out_specs=pl.BlockSpec((BR, D), lambda i: (i, 0)),
    out_shape=jax.ShapeDtypeStruct((N_PAD, D), jnp.float32),
)


def _fin_body(a_ref, yp_ref, dinv_ref, b_ref, o_ref):
    t = a_ref[0] + a_ref[1] - yp_ref[...]
    o_ref[...] = t * dinv_ref[...] + b_ref[...]


_fin_call = pl.pallas_call(
    _fin_body,
    grid=(GRID,),
    in_specs=[
        pl.BlockSpec((NC, BR, D), lambda i: (0, i, 0)),
        pl.BlockSpec((BR, D), lambda i: (i, 0)),
        pl.BlockSpec((BR, 1), lambda i: (i, 0)),
        pl.BlockSpec((1, D), lambda i: (0, 0)),
    ],
    out_specs=pl.BlockSpec((BR, D), lambda i: (i, 0)),
    out_shape=jax.ShapeDtypeStruct((N_PAD, D), jnp.float32),
)


def kernel(x, edge_index, W1, b1, W2, b2, W3, b3):
    src = edge_index[0].astype(jnp.int32)
    dst = edge_index[1].astype(jnp.int32)
    npad = E_PAD - E
    pad_src = jnp.arange(npad, dtype=jnp.int32) % N_PAD
    pad_dst = N + jnp.arange(npad, dtype=jnp.int32) % (N_PAD - N)
    src_all = jnp.concatenate([src, pad_src]).reshape(NW, EC, 1, CHUNK)
    dst_all = jnp.concatenate([dst, pad_dst]).reshape(NW, EC, 1, CHUNK)
    ncp = NW * (ECP - EC) * CHUNK
    padc = jnp.stack(
        [(jnp.arange(ncp, dtype=jnp.int32) % N_PAD).reshape(
            NW, ECP - EC, 1, CHUNK),
         (N + jnp.arange(ncp, dtype=jnp.int32) % (N_PAD - N)).reshape(
            NW, ECP - EC, 1, CHUNK)], axis=2)
    eip = jnp.concatenate(
        [jnp.stack([src_all, dst_all], axis=2), padc], axis=1)
    xp = jnp.pad(x, ((0, N_PAD - N), (0, 0)))
    zeros = jnp.zeros((N_PAD,), jnp.float32)

    degp = _deg_kernel(eip, zeros)
    dinv = _prep_call(degp.reshape(NC, N_PAD // 128, 128)).reshape(N_PAD, 1)

    b1r, b2r, b3r = (b.reshape(1, D) for b in (b1, b2, b3))
    y1 = _y1_call(xp, dinv, W1)
    a1 = _agg_kernel(y1, eip)
    y2 = _mid_call(a1, y1, dinv, b1r, W2)
    a2 = _agg_kernel(y2, eip)
    y3 = _mid_call(a2, y2, dinv, b2r, W3)
    a3 = _agg_kernel(y3, eip)
    out = _fin_call(a3, y3, dinv, b3r)
    return out[:N]


# speedup vs baseline: 3.4645x; 1.2078x over previous
import functools

import jax
import jax.numpy as jnp
from jax import lax
from jax.experimental import pallas as pl
from jax.experimental.pallas import tpu as pltpu
from jax.experimental.pallas import tpu_sc as plsc

N = 10000
D = 128
E = 320000
NC, NS = 2, 16
NW = NC * NS
N_PAD = 10240
RPT = N_PAD // NS
CHUNK = 160
EC = 64
ECP = EC + 4
E_PAD = NW * EC * CHUNK

_mesh = plsc.VectorSubcoreMesh(core_axis_name="c", subcore_axis_name="s")


@functools.partial(
    pl.kernel,
    mesh=_mesh,
    out_type=jax.ShapeDtypeStruct((NC, N_PAD), jnp.float32),
    scratch_types=[
        pltpu.VMEM((ECP, 2, 1, CHUNK), jnp.int32),
        pltpu.VMEM((1, CHUNK), jnp.float32),
        pltpu.VMEM_SHARED((N_PAD,), jnp.float32),
    ],
)
def _deg_kernel(ei_hbm, zeros_hbm, out_hbm, idx_v, ones_v, acc):
    cid = lax.axis_index("c")
    sid = lax.axis_index("s")
    wid = cid * NS + sid
    for i in range(CHUNK // 16):
        ones_v[0, pl.ds(i * 16, 16)] = jnp.ones((16,), jnp.float32)
    pltpu.sync_copy(zeros_hbm.at[pl.ds(sid * RPT, RPT)],
                    acc.at[pl.ds(sid * RPT, RPT)])
    pltpu.sync_copy(ei_hbm.at[wid], idx_v)
    plsc.subcore_barrier()

    def body(j, c):
        pltpu.sync_copy(ones_v.at[0], acc.at[idx_v.at[j, 1, 0]], add=True)
        return c

    lax.fori_loop(0, EC, body, 0)
    plsc.subcore_barrier()
    pltpu.sync_copy(acc.at[pl.ds(sid * RPT, RPT)],
                    out_hbm.at[cid, pl.ds(sid * RPT, RPT)])


@functools.partial(
    pl.kernel,
    mesh=_mesh,
    out_type=jax.ShapeDtypeStruct((NC, N_PAD, D), jnp.float32),
    scratch_types=[
        pltpu.VMEM((4, 2, 1, CHUNK), jnp.int32),
        pltpu.VMEM((2, CHUNK, D), jnp.float32),
        pltpu.VMEM_SHARED((N_PAD, D), jnp.float32),
        pltpu.SemaphoreType.DMA,
        pltpu.SemaphoreType.DMA,
        pltpu.SemaphoreType.DMA,
        pltpu.SemaphoreType.DMA,
        pltpu.SemaphoreType.DMA,
        pltpu.SemaphoreType.DMA,
    ],
)
def _agg_kernel(y_hbm, ei_hbm, out_hbm, ring, rows_v, acc,
                isem0, isem1, isem2, isem3, gsem0, gsem1):
    cid = lax.axis_index("c")
    sid = lax.axis_index("s")
    wid = cid * NS + sid
    isems = (isem0, isem1, isem2, isem3)
    gsems = (gsem0, gsem1)
    pltpu.sync_copy(y_hbm.at[pl.ds(sid * RPT, RPT)],
                    acc.at[pl.ds(sid * RPT, RPT)])
    plsc.subcore_barrier()

    for s in range(4):
        pltpu.async_copy(ei_hbm.at[wid, s], ring.at[s], isems[s])
    for s in range(2):
        pltpu.make_async_copy(ei_hbm.at[wid, s], ring.at[s], isems[s]).wait()
        pltpu.async_copy(y_hbm.at[ring.at[s, 0, 0]], rows_v.at[s], gsems[s])

    def body(i, c):
        for b in range(4):
            j = 4 * i + b
            p = b & 1
            b2 = (b + 2) % 4
            pltpu.make_async_copy(y_hbm.at[ring.at[b, 0, 0]], rows_v.at[p],
                                  gsems[p]).wait()
            pltpu.sync_copy(rows_v.at[p], acc.at[ring.at[b, 1, 0]], add=True)
            pltpu.async_copy(ei_hbm.at[wid, j + 4], ring.at[b], isems[b])
            pltpu.make_async_copy(ei_hbm.at[wid, j + 2], ring.at[b2],
                                  isems[b2]).wait()
            pltpu.async_copy(y_hbm.at[ring.at[b2, 0, 0]], rows_v.at[p],
                             gsems[p])
        return c

    lax.fori_loop(0, EC // 4, body, 0)
    pltpu.make_async_copy(y_hbm.at[ring.at[0, 0, 0]], rows_v.at[0],
                          gsem0).wait()
    pltpu.make_async_copy(y_hbm.at[ring.at[1, 0, 0]], rows_v.at[1],
                          gsem1).wait()
    pltpu.make_async_copy(ei_hbm.at[wid, 0], ring.at[2], isem2).wait()
    pltpu.make_async_copy(ei_hbm.at[wid, 0], ring.at[3], isem3).wait()
    plsc.subcore_barrier()
    pltpu.sync_copy(acc.at[pl.ds(sid * RPT, RPT)],
                    out_hbm.at[cid, pl.ds(sid * RPT, RPT)])


def _prep_body(degp_ref, o_ref):
    deg = degp_ref[0] + degp_ref[1] + 1.0
    dv = lax.rsqrt(deg)
    row = lax.broadcasted_iota(jnp.int32, (N_PAD // 128, 128), 0)
    col = lax.broadcasted_iota(jnp.int32, (N_PAD // 128, 128), 1)
    o_ref[...] = jnp.where(row * 128 + col < N, dv, 0.0)


_prep_call = pl.pallas_call(
    _prep_body,
    out_shape=jax.ShapeDtypeStruct((N_PAD // 128, 128), jnp.float32),
)

BR = 256
GRID = N_PAD // BR


def _y1_body(x_ref, dinv_ref, w_ref, o_ref):
    o_ref[...] = jnp.dot(x_ref[...] * dinv_ref[...], w_ref[...],
                         preferred_element_type=jnp.float32)


_y1_call = pl.pallas_call(
    _y1_body,
    grid=(GRID,),
    in_specs=[
        pl.BlockSpec((BR, D), lambda i: (i, 0)),
        pl.BlockSpec((BR, 1), lambda i: (i, 0)),
        pl.BlockSpec((D, D), lambda i: (0, 0)),
    ],
    out_specs=pl.BlockSpec((BR, D), lambda i: (i, 0)),
    out_shape=jax.ShapeDtypeStruct((N_PAD, D), jnp.float32),
)


def _mid_body(a_ref, yp_ref, dinv_ref, b_ref, w_ref, o_ref):
    t = a_ref[0] + a_ref[1] - yp_ref[...]
    t = t * dinv_ref[...] + b_ref[...]
    t = jnp.maximum(t, 0.0)
    o_ref[...] = jnp.dot(t * dinv_ref[...], w_ref[...],
                         preferred_element_type=jnp.float32)


_mid_call = pl.pallas_call(
    _mid_body,
    grid=(GRID,),
    in_specs=[
        pl.BlockSpec((NC, BR, D), lambda i: (0, i, 0)),
        pl.BlockSpec((BR, D), lambda i: (i, 0)),
        pl.BlockSpec((BR, 1), lambda i: (i, 0)),
        pl.BlockSpec((1, D), lambda i: (0, 0)),
        pl.BlockSpec((D, D), lambda i: (0, 0)),
    ],
    out_specs=pl.BlockSpec((BR, D), lambda i: (i, 0)),
    out_shape=jax.ShapeDtypeStruct((N_PAD, D), jnp.float32),
)


def _fin_body(a_ref, yp_ref, dinv_ref, b_ref, o_ref):
    t = a_ref[0] + a_ref[1] - yp_ref[...]
    o_ref[...] = t * dinv_ref[...] + b_ref[...]


_fin_call = pl.pallas_call(
    _fin_body,
    grid=(GRID,),
    in_specs=[
        pl.BlockSpec((NC, BR, D), lambda i: (0, i, 0)),
        pl.BlockSpec((BR, D), lambda i: (i, 0)),
        pl.BlockSpec((BR, 1), lambda i: (i, 0)),
        pl.BlockSpec((1, D), lambda i: (0, 0)),
    ],
    out_specs=pl.BlockSpec((BR, D), lambda i: (i, 0)),
    out_shape=jax.ShapeDtypeStruct((N_PAD, D), jnp.float32),
)


def kernel(x, edge_index, W1, b1, W2, b2, W3, b3):
    src = edge_index[0].astype(jnp.int32)
    dst = edge_index[1].astype(jnp.int32)
    npad = E_PAD - E
    pad_src = jnp.arange(npad, dtype=jnp.int32) % N_PAD
    pad_dst = N + jnp.arange(npad, dtype=jnp.int32) % (N_PAD - N)
    src_all = jnp.concatenate([src, pad_src]).reshape(NW, EC, 1, CHUNK)
    dst_all = jnp.concatenate([dst, pad_dst]).reshape(NW, EC, 1, CHUNK)
    ncp = NW * (ECP - EC) * CHUNK
    padc = jnp.stack(
        [(jnp.arange(ncp, dtype=jnp.int32) % N_PAD).reshape(
            NW, ECP - EC, 1, CHUNK),
         (N + jnp.arange(ncp, dtype=jnp.int32) % (N_PAD - N)).reshape(
            NW, ECP - EC, 1, CHUNK)], axis=2)
    eip = jnp.concatenate(
        [jnp.stack([src_all, dst_all], axis=2), padc], axis=1)
    xp = jnp.pad(x, ((0, N_PAD - N), (0, 0)))
    zeros = jnp.zeros((N_PAD,), jnp.float32)

    degp = _deg_kernel(eip, zeros)
    dinv = _prep_call(degp.reshape(NC, N_PAD // 128, 128)).reshape(N_PAD, 1)

    b1r, b2r, b3r = (b.reshape(1, D) for b in (b1, b2, b3))
    y1 = _y1_call(xp, dinv, W1)
    a1 = _agg_kernel(y1, eip)
    y2 = _mid_call(a1, y1, dinv, b1r, W2)
    a2 = _agg_kernel(y2, eip)
    y3 = _mid_call(a2, y2, dinv, b2r, W3)
    a3 = _agg_kernel(y3, eip)
    out = _fin_call(a3, y3, dinv, b3r)
    return out[:N]
